# Initial kernel scaffold; baseline (speedup 1.0000x reference)
import jax
import jax.numpy as jnp
from jax.experimental import pallas as pl


def kernel(x, src0, dst0, src1, dst1, Wl0, Wr0, b0, a0, Wl1, Wr1, b1, a1):
    raise NotImplementedError("write your pallas kernel here")


# speedup vs baseline: 14.0320x; 14.0320x over previous
import functools

import jax
import jax.numpy as jnp
from jax import lax
from jax.experimental import pallas as pl
from jax.experimental.pallas import tpu as pltpu
from jax.experimental.pallas import tpu_sc as plsc

N_NODES = 10000
D = 128
N1 = 2048
N2 = 1024
E0 = 320000
E1 = 65536
CW = 16

NC = 2
NS = 16
NW = NC * NS


def _make_sc_agg(n_tgt, n_edges, k):
    chunks_total = n_edges // k
    assert chunks_total * k == n_edges
    chunks_per_w = chunks_total // NW
    assert chunks_per_w * NW == chunks_total and chunks_per_w % 8 == 0
    edges_per_w = n_edges // NW
    assert edges_per_w % CW == 0
    rows_per_tile = n_tgt // NS

    mesh = plsc.VectorSubcoreMesh(core_axis_name="c", subcore_axis_name="s")

    @functools.partial(
        pl.kernel,
        out_type=(
            jax.ShapeDtypeStruct((NC, n_tgt, D), jnp.float32),
            jax.ShapeDtypeStruct((NW, n_tgt * CW), jnp.float32),
        ),
        mesh=mesh,
        compiler_params=pltpu.CompilerParams(needs_layout_passes=False),
        scratch_types=[
            pltpu.VMEM((chunks_per_w, k), jnp.int32),
            pltpu.VMEM((chunks_per_w, k), jnp.int32),
            pltpu.VMEM((edges_per_w,), jnp.int32),
            pltpu.VMEM((2, k, D), jnp.float32),
            pltpu.VMEM((n_tgt * CW,), jnp.float32),
            pltpu.VMEM_SHARED((n_tgt, D), jnp.float32),
            pltpu.SemaphoreType.DMA,
            pltpu.SemaphoreType.DMA,
        ],
    )
    def sc_kernel(src_hbm, dst_hbm, dstf_hbm, x_hbm, zrows_hbm, zcnt_hbm,
                  agg_out, cnt_out,
                  src_v, dst_v, dstf_v, rows_v, cnt_v, agg_sh, sem0, sem1):
        cid = lax.axis_index("c")
        sid = lax.axis_index("s")
        wid = sid * NC + cid
        base_chunk = wid * chunks_per_w
        sems = (sem0, sem1)

        pltpu.sync_copy(src_hbm.at[pl.ds(base_chunk, chunks_per_w)], src_v)
        pltpu.sync_copy(dst_hbm.at[pl.ds(base_chunk, chunks_per_w)], dst_v)
        pltpu.sync_copy(dstf_hbm.at[pl.ds(wid * edges_per_w, edges_per_w)],
                        dstf_v)
        pltpu.sync_copy(zcnt_hbm, cnt_v)

        r0 = sid * rows_per_tile
        pltpu.sync_copy(zrows_hbm.at[pl.ds(r0, rows_per_tile)],
                        agg_sh.at[pl.ds(r0, rows_per_tile)])
        plsc.subcore_barrier()

        for b in range(2):
            pltpu.async_copy(x_hbm.at[src_v.at[b]], rows_v.at[b], sems[b])

        lanes = lax.broadcasted_iota(jnp.int32, (CW,), 0)
        ones16 = jnp.ones((CW,), jnp.float32)

        @pl.loop(0, edges_per_w // CW)
        def _count(j):
            d16 = dstf_v[pl.ds(j * CW, CW)]
            plsc.addupdate_scatter(cnt_v, [d16 * CW + lanes], ones16)

        @pl.loop(0, chunks_per_w, step=2)
        def _chunks(c):
            for b in range(2):
                cc = c + b
                pltpu.make_async_copy(
                    x_hbm.at[src_v.at[cc]], rows_v.at[b], sems[b]).wait()
                pltpu.sync_copy(rows_v.at[b], agg_sh.at[dst_v.at[cc]],
                                add=True)

                @pl.when(cc + 2 < chunks_per_w)
                def _():
                    pltpu.async_copy(
                        x_hbm.at[src_v.at[cc + 2]], rows_v.at[b], sems[b])

        pltpu.sync_copy(cnt_v, cnt_out.at[wid])

        plsc.subcore_barrier()
        pltpu.sync_copy(agg_sh.at[pl.ds(r0, rows_per_tile)],
                        agg_out.at[cid, pl.ds(r0, rows_per_tile)])

    return sc_kernel


def _tc_layer(aggp, cntp, x_tgt, Wl, Wr, b, a):
    n_tgt = aggp.shape[1]

    def body(agg_ref, cnt_ref, x_ref, wl_ref, wr_ref, b_ref, a_ref, o_ref):
        agg = agg_ref[0] + agg_ref[1]
        cnt = jnp.sum(cnt_ref[...].reshape(NW, n_tgt, CW), axis=(0, 2))[:, None]
        mean = agg / jnp.maximum(cnt, 1.0)
        h = jnp.dot(mean, wl_ref[...], preferred_element_type=jnp.float32)
        h = h + jnp.dot(x_ref[...], wr_ref[...],
                        preferred_element_type=jnp.float32)
        h = h + b_ref[...]
        o_ref[...] = jnp.where(h > 0, h, a_ref[...] * h)

    return pl.pallas_call(
        body,
        out_shape=jax.ShapeDtypeStruct((n_tgt, D), jnp.float32),
    )(aggp, cntp, x_tgt, Wl, Wr, b.reshape(1, D), a.reshape(1, D))


K0 = 125
K1 = 64

_sc_agg0 = _make_sc_agg(N1, E0, K0)
_sc_agg1 = _make_sc_agg(N2, E1, K1)


def kernel(x, src0, dst0, src1, dst1, Wl0, Wr0, b0, a0, Wl1, Wr1, b1, a1):
    src0 = src0.astype(jnp.int32)
    dst0 = dst0.astype(jnp.int32)
    src1 = src1.astype(jnp.int32)
    dst1 = dst1.astype(jnp.int32)

    z0 = jnp.zeros((N1, D), jnp.float32)
    zc0 = jnp.zeros((N1 * CW,), jnp.float32)
    z1 = jnp.zeros((N2, D), jnp.float32)
    zc1 = jnp.zeros((N2 * CW,), jnp.float32)

    agg0, cnt0 = _sc_agg0(src0.reshape(-1, K0), dst0.reshape(-1, K0),
                          dst0, x, z0, zc0)
    h = _tc_layer(agg0, cnt0, x[:N1], Wl0, Wr0, b0, a0)
    agg1, cnt1 = _sc_agg1(src1.reshape(-1, K1), dst1.reshape(-1, K1),
                          dst1, h, z1, zc1)
    out = _tc_layer(agg1, cnt1, h[:N2], Wl1, Wr1, b1, a1)
    return out


# speedup vs baseline: 14.6049x; 1.0408x over previous
import functools

import jax
import jax.numpy as jnp
from jax import lax
from jax.experimental import pallas as pl
from jax.experimental.pallas import tpu as pltpu
from jax.experimental.pallas import tpu_sc as plsc

N_NODES = 10000
D = 128
N1 = 2048
N2 = 1024
E0 = 320000
E1 = 65536
CW = 16

NC = 2
NS = 16
NW = NC * NS


def _make_sc_agg(n_tgt, n_edges, k, flat_counts):
    chunks_total = n_edges // k
    assert chunks_total * k == n_edges
    chunks_per_w = chunks_total // NW
    assert chunks_per_w * NW == chunks_total and chunks_per_w % 8 == 0
    edges_per_w = n_edges // NW
    assert edges_per_w % CW == 0
    rows_per_tile = n_tgt // NS
    if flat_counts:
        n_cvecs = edges_per_w // CW
        cpc = -(-n_cvecs // chunks_per_w)
    else:
        assert k % CW == 0
        cpc = k // CW

    mesh = plsc.VectorSubcoreMesh(core_axis_name="c", subcore_axis_name="s")

    scratch = [
        pltpu.VMEM((chunks_per_w, k), jnp.int32),
        pltpu.VMEM((chunks_per_w, k), jnp.int32),
        pltpu.VMEM((edges_per_w,), jnp.int32) if flat_counts else None,
        pltpu.VMEM((2, k, D), jnp.float32),
        pltpu.VMEM((n_tgt * CW,), jnp.float32),
        pltpu.VMEM_SHARED((n_tgt, D), jnp.float32),
        pltpu.SemaphoreType.DMA,
        pltpu.SemaphoreType.DMA,
        pltpu.SemaphoreType.DMA,
    ]
    scratch = [s for s in scratch if s is not None]

    def body(src_hbm, dst_hbm, dstf_hbm, x_hbm, zrows_hbm, zcnt_hbm,
             agg_out, cnt_out,
             src_v, dst_v, dstf_v, rows_v, cnt_v, agg_sh, sem0, sem1, sem_s):
        cid = lax.axis_index("c")
        sid = lax.axis_index("s")
        wid = sid * NC + cid
        base_chunk = wid * chunks_per_w
        sems = (sem0, sem1)

        pltpu.sync_copy(src_hbm.at[pl.ds(base_chunk, chunks_per_w)], src_v)
        pltpu.sync_copy(dst_hbm.at[pl.ds(base_chunk, chunks_per_w)], dst_v)
        if flat_counts:
            pltpu.sync_copy(
                dstf_hbm.at[pl.ds(wid * edges_per_w, edges_per_w)], dstf_v)
        pltpu.sync_copy(zcnt_hbm, cnt_v)

        r0 = sid * rows_per_tile
        pltpu.sync_copy(zrows_hbm.at[pl.ds(r0, rows_per_tile)],
                        agg_sh.at[pl.ds(r0, rows_per_tile)])
        plsc.subcore_barrier()

        for b in range(2):
            pltpu.async_copy(x_hbm.at[src_v.at[b]], rows_v.at[b], sems[b])

        lanes = lax.broadcasted_iota(jnp.int32, (CW,), 0)
        ones16 = jnp.ones((CW,), jnp.float32)

        def count_vec(d16):
            plsc.addupdate_scatter(cnt_v, [d16 * CW + lanes], ones16)

        @pl.loop(0, chunks_per_w, step=2)
        def _chunks(c):
            for b in range(2):
                cc = c + b
                pltpu.make_async_copy(
                    x_hbm.at[src_v.at[cc]], rows_v.at[b], sems[b]).wait()
                sdesc = pltpu.async_copy(
                    rows_v.at[b], agg_sh.at[dst_v.at[cc]], sem_s, add=True)

                for j in range(cpc):
                    if flat_counts:
                        v = cc * cpc + j

                        @pl.when(v < n_cvecs)
                        def _():
                            count_vec(dstf_v[pl.ds(v * CW, CW)])
                    else:
                        count_vec(dst_v[cc, pl.ds(j * CW, CW)])

                sdesc.wait()

                @pl.when(cc + 2 < chunks_per_w)
                def _():
                    pltpu.async_copy(
                        x_hbm.at[src_v.at[cc + 2]], rows_v.at[b], sems[b])

        pltpu.sync_copy(cnt_v, cnt_out.at[wid])

        plsc.subcore_barrier()
        pltpu.sync_copy(agg_sh.at[pl.ds(r0, rows_per_tile)],
                        agg_out.at[cid, pl.ds(r0, rows_per_tile)])

    if not flat_counts:
        full = body

        def body_no_flat(src_hbm, dst_hbm, x_hbm, zrows_hbm, zcnt_hbm,
                         agg_out, cnt_out,
                         src_v, dst_v, rows_v, cnt_v, agg_sh,
                         sem0, sem1, sem_s):
            full(src_hbm, dst_hbm, None, x_hbm, zrows_hbm, zcnt_hbm,
                 agg_out, cnt_out,
                 src_v, dst_v, None, rows_v, cnt_v, agg_sh,
                 sem0, sem1, sem_s)

        fn = body_no_flat
    else:
        fn = body

    return pl.kernel(
        fn,
        out_type=(
            jax.ShapeDtypeStruct((NC, n_tgt, D), jnp.float32),
            jax.ShapeDtypeStruct((NW, n_tgt * CW), jnp.float32),
        ),
        mesh=mesh,
        compiler_params=pltpu.CompilerParams(needs_layout_passes=False),
        scratch_types=scratch,
    )


def _tc_layer(aggp, cntp, x_tgt, Wl, Wr, b, a):
    n_tgt = aggp.shape[1]

    def body(agg_ref, cnt_ref, x_ref, wl_ref, wr_ref, b_ref, a_ref, o_ref):
        agg = agg_ref[0] + agg_ref[1]
        cnt = jnp.sum(cnt_ref[...].reshape(NW, n_tgt, CW), axis=(0, 2))[:, None]
        mean = agg / jnp.maximum(cnt, 1.0)
        h = jnp.dot(mean, wl_ref[...], preferred_element_type=jnp.float32)
        h = h + jnp.dot(x_ref[...], wr_ref[...],
                        preferred_element_type=jnp.float32)
        h = h + b_ref[...]
        o_ref[...] = jnp.where(h > 0, h, a_ref[...] * h)

    return pl.pallas_call(
        body,
        out_shape=jax.ShapeDtypeStruct((n_tgt, D), jnp.float32),
    )(aggp, cntp, x_tgt, Wl, Wr, b.reshape(1, D), a.reshape(1, D))


K0 = 125
K1 = 128

_sc_agg0 = _make_sc_agg(N1, E0, K0, flat_counts=True)
_sc_agg1 = _make_sc_agg(N2, E1, K1, flat_counts=False)


def kernel(x, src0, dst0, src1, dst1, Wl0, Wr0, b0, a0, Wl1, Wr1, b1, a1):
    src0 = src0.astype(jnp.int32)
    dst0 = dst0.astype(jnp.int32)
    src1 = src1.astype(jnp.int32)
    dst1 = dst1.astype(jnp.int32)

    z0 = jnp.zeros((N1, D), jnp.float32)
    zc0 = jnp.zeros((N1 * CW,), jnp.float32)
    z1 = jnp.zeros((N2, D), jnp.float32)
    zc1 = jnp.zeros((N2 * CW,), jnp.float32)

    agg0, cnt0 = _sc_agg0(src0.reshape(-1, K0), dst0.reshape(-1, K0),
                          dst0, x, z0, zc0)
    h = _tc_layer(agg0, cnt0, x[:N1], Wl0, Wr0, b0, a0)
    agg1, cnt1 = _sc_agg1(src1.reshape(-1, K1), dst1.reshape(-1, K1),
                          h, z1, zc1)
    out = _tc_layer(agg1, cnt1, h[:N2], Wl1, Wr1, b1, a1)
    return out


# speedup vs baseline: 14.9571x; 1.0241x over previous
import functools

import jax
import jax.numpy as jnp
from jax import lax
from jax.experimental import pallas as pl
from jax.experimental.pallas import tpu as pltpu
from jax.experimental.pallas import tpu_sc as plsc

N_NODES = 10000
D = 128
N1 = 2048
N2 = 1024
E0 = 320000
E1 = 65536
CW = 16

NC = 2
NS = 16
NW = NC * NS


def _make_sc_agg(n_tgt, n_edges, k, flat_counts):
    chunks_total = n_edges // k
    assert chunks_total * k == n_edges
    chunks_per_w = chunks_total // NW
    assert chunks_per_w * NW == chunks_total and chunks_per_w % 8 == 0
    edges_per_w = n_edges // NW
    assert edges_per_w % CW == 0
    rows_per_tile = n_tgt // NS
    if flat_counts:
        n_cvecs = edges_per_w // CW
        cpc = -(-n_cvecs // chunks_per_w)
    else:
        assert k % CW == 0
        cpc = k // CW

    mesh = plsc.VectorSubcoreMesh(core_axis_name="c", subcore_axis_name="s")

    scratch = [
        pltpu.VMEM((chunks_per_w, k), jnp.int32),
        pltpu.VMEM((chunks_per_w, k), jnp.int32),
        pltpu.VMEM((edges_per_w,), jnp.int32) if flat_counts else None,
        pltpu.VMEM((2, k, D), jnp.float32),
        pltpu.VMEM((n_tgt * CW,), jnp.float32),
        pltpu.VMEM_SHARED((n_tgt, D), jnp.float32),
        [pltpu.SemaphoreType.DMA] * 2,
        [pltpu.SemaphoreType.DMA] * 2,
    ]
    scratch = [s for s in scratch if s is not None]

    def body(src_hbm, dst_hbm, dstf_hbm, x_hbm, zrows_hbm, zcnt_hbm,
             agg_out, cnt_out,
             src_v, dst_v, dstf_v, rows_v, cnt_v, agg_sh, sem_g, sem_s):
        cid = lax.axis_index("c")
        sid = lax.axis_index("s")
        wid = sid * NC + cid
        base_chunk = wid * chunks_per_w

        d_src = pltpu.async_copy(
            src_hbm.at[pl.ds(base_chunk, chunks_per_w)], src_v, sem_g[0])
        d_dst = pltpu.async_copy(
            dst_hbm.at[pl.ds(base_chunk, chunks_per_w)], dst_v, sem_g[1])
        if flat_counts:
            d_dstf = pltpu.async_copy(
                dstf_hbm.at[pl.ds(wid * edges_per_w, edges_per_w)], dstf_v,
                sem_s[0])
        d_zc = pltpu.async_copy(zcnt_hbm, cnt_v, sem_s[1])

        r0 = sid * rows_per_tile
        pltpu.sync_copy(zrows_hbm.at[pl.ds(r0, rows_per_tile)],
                        agg_sh.at[pl.ds(r0, rows_per_tile)])

        d_src.wait()
        d_dst.wait()
        for b in range(2):
            pltpu.async_copy(x_hbm.at[src_v.at[b]], rows_v.at[b], sem_g[b])
        if flat_counts:
            d_dstf.wait()
        d_zc.wait()
        plsc.subcore_barrier()

        lanes = lax.broadcasted_iota(jnp.int32, (CW,), 0)
        ones16 = jnp.ones((CW,), jnp.float32)

        def count_vec(d16):
            plsc.addupdate_scatter(cnt_v, [d16 * CW + lanes], ones16)

        @pl.loop(0, chunks_per_w, step=2)
        def _chunks(c):
            for b in range(2):
                cc = c + b
                pltpu.make_async_copy(
                    x_hbm.at[src_v.at[cc]], rows_v.at[b], sem_g[b]).wait()
                sdesc = pltpu.async_copy(
                    rows_v.at[b], agg_sh.at[dst_v.at[cc]], sem_s[b],
                    add=True)

                for j in range(cpc):
                    if flat_counts:
                        v = cc * cpc + j

                        @pl.when(v < n_cvecs)
                        def _():
                            count_vec(dstf_v[pl.ds(v * CW, CW)])
                    else:
                        count_vec(dst_v[cc, pl.ds(j * CW, CW)])

                sdesc.wait()

                @pl.when(cc + 2 < chunks_per_w)
                def _():
                    pltpu.async_copy(
                        x_hbm.at[src_v.at[cc + 2]], rows_v.at[b], sem_g[b])

        pltpu.sync_copy(cnt_v, cnt_out.at[wid])

        plsc.subcore_barrier()
        pltpu.sync_copy(agg_sh.at[pl.ds(r0, rows_per_tile)],
                        agg_out.at[cid, pl.ds(r0, rows_per_tile)])

    if not flat_counts:
        full = body

        def body_no_flat(src_hbm, dst_hbm, x_hbm, zrows_hbm, zcnt_hbm,
                         agg_out, cnt_out,
                         src_v, dst_v, rows_v, cnt_v, agg_sh, sem_g, sem_s):
            full(src_hbm, dst_hbm, None, x_hbm, zrows_hbm, zcnt_hbm,
                 agg_out, cnt_out,
                 src_v, dst_v, None, rows_v, cnt_v, agg_sh, sem_g, sem_s)

        fn = body_no_flat
    else:
        fn = body

    return pl.kernel(
        fn,
        out_type=(
            jax.ShapeDtypeStruct((NC, n_tgt, D), jnp.float32),
            jax.ShapeDtypeStruct((NW, n_tgt * CW), jnp.float32),
        ),
        mesh=mesh,
        compiler_params=pltpu.CompilerParams(needs_layout_passes=False),
        scratch_types=scratch,
    )


def _tc_layer(aggp, cntp, x_tgt, Wl, Wr, b, a):
    n_tgt = aggp.shape[1]

    def body(agg_ref, cnt_ref, x_ref, wl_ref, wr_ref, b_ref, a_ref, o_ref):
        agg = agg_ref[0] + agg_ref[1]
        cnt = jnp.sum(cnt_ref[...].reshape(NW, n_tgt, CW), axis=(0, 2))[:, None]
        mean = agg / jnp.maximum(cnt, 1.0)
        h = jnp.dot(mean, wl_ref[...], preferred_element_type=jnp.float32)
        h = h + jnp.dot(x_ref[...], wr_ref[...],
                        preferred_element_type=jnp.float32)
        h = h + b_ref[...]
        o_ref[...] = jnp.where(h > 0, h, a_ref[...] * h)

    return pl.pallas_call(
        body,
        out_shape=jax.ShapeDtypeStruct((n_tgt, D), jnp.float32),
    )(aggp, cntp, x_tgt, Wl, Wr, b.reshape(1, D), a.reshape(1, D))


K0 = 125
K1 = 128

_sc_agg0 = _make_sc_agg(N1, E0, K0, flat_counts=True)
_sc_agg1 = _make_sc_agg(N2, E1, K1, flat_counts=False)


def kernel(x, src0, dst0, src1, dst1, Wl0, Wr0, b0, a0, Wl1, Wr1, b1, a1):
    src0 = src0.astype(jnp.int32)
    dst0 = dst0.astype(jnp.int32)
    src1 = src1.astype(jnp.int32)
    dst1 = dst1.astype(jnp.int32)

    z0 = jnp.zeros((N1, D), jnp.float32)
    zc0 = jnp.zeros((N1 * CW,), jnp.float32)
    z1 = jnp.zeros((N2, D), jnp.float32)
    zc1 = jnp.zeros((N2 * CW,), jnp.float32)

    agg0, cnt0 = _sc_agg0(src0.reshape(-1, K0), dst0.reshape(-1, K0),
                          dst0, x, z0, zc0)
    h = _tc_layer(agg0, cnt0, x[:N1], Wl0, Wr0, b0, a0)
    agg1, cnt1 = _sc_agg1(src1.reshape(-1, K1), dst1.reshape(-1, K1),
                          h, z1, zc1)
    out = _tc_layer(agg1, cnt1, h[:N2], Wl1, Wr1, b1, a1)
    return out


# speedup vs baseline: 15.1364x; 1.0120x over previous
import functools

import jax
import jax.numpy as jnp
from jax import lax
from jax.experimental import pallas as pl
from jax.experimental.pallas import tpu as pltpu
from jax.experimental.pallas import tpu_sc as plsc

N_NODES = 10000
D = 128
N1 = 2048
N2 = 1024
E0 = 320000
E1 = 65536
CW = 16

NC = 2
NS = 16
NW = NC * NS


def _make_sc_agg(n_tgt, n_edges, k, flat_counts):
    chunks_total = n_edges // k
    assert chunks_total * k == n_edges
    chunks_per_w = chunks_total // NW
    assert chunks_per_w * NW == chunks_total and chunks_per_w % 8 == 0
    edges_per_w = n_edges // NW
    assert edges_per_w % CW == 0
    rows_per_tile = n_tgt // NS
    if flat_counts:
        n_cvecs = edges_per_w // CW
        cpc = -(-n_cvecs // chunks_per_w)
    else:
        assert k % CW == 0
        cpc = k // CW

    mesh = plsc.VectorSubcoreMesh(core_axis_name="c", subcore_axis_name="s")

    scratch = [
        pltpu.VMEM((chunks_per_w, k), jnp.int32),
        pltpu.VMEM((chunks_per_w, k), jnp.int32),
        pltpu.VMEM((edges_per_w,), jnp.int32) if flat_counts else None,
        pltpu.VMEM((2, k, D), jnp.float32),
        pltpu.VMEM((n_tgt * CW,), jnp.float32),
        pltpu.VMEM_SHARED((n_tgt, D), jnp.float32),
        [pltpu.SemaphoreType.DMA] * 2,
        [pltpu.SemaphoreType.DMA] * 2,
    ]
    scratch = [s for s in scratch if s is not None]

    def body(src_hbm, dst_hbm, dstf_hbm, x_hbm, zrows_hbm, zcnt_hbm,
             agg_out, cnt_out,
             src_v, dst_v, dstf_v, rows_v, cnt_v, agg_sh, sem_g, sem_s):
        cid = lax.axis_index("c")
        sid = lax.axis_index("s")
        wid = sid * NC + cid
        base_chunk = wid * chunks_per_w

        d_src = pltpu.async_copy(
            src_hbm.at[pl.ds(base_chunk, chunks_per_w)], src_v, sem_g[0])
        d_dst = pltpu.async_copy(
            dst_hbm.at[pl.ds(base_chunk, chunks_per_w)], dst_v, sem_g[1])
        if flat_counts:
            d_dstf = pltpu.async_copy(
                dstf_hbm.at[pl.ds(wid * edges_per_w, edges_per_w)], dstf_v,
                sem_s[0])
        d_zc = pltpu.async_copy(zcnt_hbm, cnt_v, sem_s[1])

        r0 = sid * rows_per_tile
        pltpu.sync_copy(zrows_hbm.at[pl.ds(r0, rows_per_tile)],
                        agg_sh.at[pl.ds(r0, rows_per_tile)])

        d_src.wait()
        d_dst.wait()
        for b in range(2):
            pltpu.async_copy(x_hbm.at[src_v.at[b]], rows_v.at[b], sem_g[b])
        if flat_counts:
            d_dstf.wait()
        d_zc.wait()
        plsc.subcore_barrier()

        lanes = lax.broadcasted_iota(jnp.int32, (CW,), 0)
        ones16 = jnp.ones((CW,), jnp.float32)

        def count_vec(d16):
            plsc.addupdate_scatter(cnt_v, [d16 * CW + lanes], ones16)

        @pl.loop(0, chunks_per_w, step=2)
        def _chunks(c):
            for b in range(2):
                cc = c + b
                pltpu.make_async_copy(
                    x_hbm.at[src_v.at[cc]], rows_v.at[b], sem_g[b]).wait()
                sdesc = pltpu.async_copy(
                    rows_v.at[b], agg_sh.at[dst_v.at[cc]], sem_s[b],
                    add=True)

                for j in range(cpc):
                    if flat_counts:
                        v = cc * cpc + j

                        @pl.when(v < n_cvecs)
                        def _():
                            count_vec(dstf_v[pl.ds(v * CW, CW)])
                    else:
                        count_vec(dst_v[cc, pl.ds(j * CW, CW)])

                sdesc.wait()

                @pl.when(cc + 2 < chunks_per_w)
                def _():
                    pltpu.async_copy(
                        x_hbm.at[src_v.at[cc + 2]], rows_v.at[b], sem_g[b])

        pltpu.sync_copy(cnt_v, cnt_out.at[wid])

        plsc.subcore_barrier()
        pltpu.sync_copy(agg_sh.at[pl.ds(r0, rows_per_tile)],
                        agg_out.at[cid, pl.ds(r0, rows_per_tile)])

    if not flat_counts:
        full = body

        def body_no_flat(src_hbm, dst_hbm, x_hbm, zrows_hbm, zcnt_hbm,
                         agg_out, cnt_out,
                         src_v, dst_v, rows_v, cnt_v, agg_sh, sem_g, sem_s):
            full(src_hbm, dst_hbm, None, x_hbm, zrows_hbm, zcnt_hbm,
                 agg_out, cnt_out,
                 src_v, dst_v, None, rows_v, cnt_v, agg_sh, sem_g, sem_s)

        fn = body_no_flat
    else:
        fn = body

    return pl.kernel(
        fn,
        out_type=(
            jax.ShapeDtypeStruct((NC, n_tgt, D), jnp.float32),
            jax.ShapeDtypeStruct((NW, n_tgt * CW), jnp.float32),
        ),
        mesh=mesh,
        compiler_params=pltpu.CompilerParams(needs_layout_passes=False),
        scratch_types=scratch,
    )


def _tc_layer(aggp, cntp, x_tgt, Wl, Wr, b, a):
    n_tgt = aggp.shape[1]

    def body(agg_ref, cnt_ref, x_ref, wl_ref, wr_ref, b_ref, a_ref, o_ref):
        agg = agg_ref[0] + agg_ref[1]
        cnt = jnp.sum(cnt_ref[...].reshape(NW, n_tgt, CW), axis=(0, 2))[:, None]
        mean = agg / jnp.maximum(cnt, 1.0)
        h = jnp.dot(mean, wl_ref[...], preferred_element_type=jnp.float32)
        h = h + jnp.dot(x_ref[...], wr_ref[...],
                        preferred_element_type=jnp.float32)
        h = h + b_ref[...]
        o_ref[...] = jnp.where(h > 0, h, a_ref[...] * h)

    return pl.pallas_call(
        body,
        out_shape=jax.ShapeDtypeStruct((n_tgt, D), jnp.float32),
        grid=(1,),
        in_specs=[
            pl.BlockSpec((NC, n_tgt, D), lambda i: (0, 0, 0)),
            pl.BlockSpec((NW, n_tgt * CW), lambda i: (0, 0)),
            pl.BlockSpec((n_tgt, D), lambda i: (0, 0)),
            pl.BlockSpec((D, D), lambda i: (0, 0)),
            pl.BlockSpec((D, D), lambda i: (0, 0)),
            pl.BlockSpec((1, D), lambda i: (0, 0)),
            pl.BlockSpec((1, D), lambda i: (0, 0)),
        ],
        out_specs=pl.BlockSpec((n_tgt, D), lambda i: (0, 0)),
    )(aggp, cntp, x_tgt, Wl, Wr, b.reshape(1, D), a.reshape(1, D))


K0 = 125
K1 = 128

_sc_agg0 = _make_sc_agg(N1, E0, K0, flat_counts=True)
_sc_agg1 = _make_sc_agg(N2, E1, K1, flat_counts=False)


def kernel(x, src0, dst0, src1, dst1, Wl0, Wr0, b0, a0, Wl1, Wr1, b1, a1):
    src0 = src0.astype(jnp.int32)
    dst0 = dst0.astype(jnp.int32)
    src1 = src1.astype(jnp.int32)
    dst1 = dst1.astype(jnp.int32)

    z0 = jnp.zeros((N1, D), jnp.float32)
    zc0 = jnp.zeros((N1 * CW,), jnp.float32)
    z1 = jnp.zeros((N2, D), jnp.float32)
    zc1 = jnp.zeros((N2 * CW,), jnp.float32)

    agg0, cnt0 = _sc_agg0(src0.reshape(-1, K0), dst0.reshape(-1, K0),
                          dst0, x, z0, zc0)
    h = _tc_layer(agg0, cnt0, x, Wl0, Wr0, b0, a0)
    agg1, cnt1 = _sc_agg1(src1.reshape(-1, K1), dst1.reshape(-1, K1),
                          h, z1, zc1)
    out = _tc_layer(agg1, cnt1, h, Wl1, Wr1, b1, a1)
    return out


# speedup vs baseline: 15.2900x; 1.0101x over previous
import functools

import jax
import jax.numpy as jnp
from jax import lax
from jax.experimental import pallas as pl
from jax.experimental.pallas import tpu as pltpu
from jax.experimental.pallas import tpu_sc as plsc

N_NODES = 10000
D = 128
N1 = 2048
N2 = 1024
E0 = 320000
E1 = 65536
CW = 16

NC = 2
NS = 16
NW = NC * NS


def _make_sc_agg(n_tgt, n_edges, k, flat_counts):
    chunks_total = n_edges // k
    assert chunks_total * k == n_edges
    chunks_per_w = chunks_total // NW
    assert chunks_per_w * NW == chunks_total and chunks_per_w % 8 == 0
    edges_per_w = n_edges // NW
    assert edges_per_w % CW == 0
    rows_per_tile = n_tgt // NS
    if flat_counts:
        n_cvecs = edges_per_w // CW
        cpc = -(-n_cvecs // chunks_per_w)
    else:
        assert k % CW == 0
        cpc = k // CW

    mesh = plsc.VectorSubcoreMesh(core_axis_name="c", subcore_axis_name="s")

    scratch = [
        pltpu.VMEM((chunks_per_w, k), jnp.int32),
        pltpu.VMEM((chunks_per_w, k), jnp.int32),
        pltpu.VMEM((edges_per_w,), jnp.int32) if flat_counts else None,
        pltpu.VMEM((2, k, D), jnp.float32),
        pltpu.VMEM((n_tgt * CW,), jnp.float32),
        pltpu.VMEM_SHARED((n_tgt, D), jnp.float32),
        [pltpu.SemaphoreType.DMA] * 2,
        [pltpu.SemaphoreType.DMA] * 2,
    ]
    scratch = [s for s in scratch if s is not None]

    def body(src_hbm, dst_hbm, dstf_hbm, x_hbm, zrows_hbm, zcnt_hbm,
             agg_out, cnt_out,
             src_v, dst_v, dstf_v, rows_v, cnt_v, agg_sh, sem_g, sem_s):
        cid = lax.axis_index("c")
        sid = lax.axis_index("s")
        wid = sid * NC + cid
        base_chunk = wid * chunks_per_w

        d_src = pltpu.async_copy(
            src_hbm.at[pl.ds(base_chunk, chunks_per_w)], src_v, sem_g[0])
        d_dst = pltpu.async_copy(
            dst_hbm.at[pl.ds(base_chunk, chunks_per_w)], dst_v, sem_g[1])
        if flat_counts:
            d_dstf = pltpu.async_copy(
                dstf_hbm.at[pl.ds(wid * edges_per_w, edges_per_w)], dstf_v,
                sem_s[0])
        d_zc = pltpu.async_copy(zcnt_hbm, cnt_v, sem_s[1])

        r0 = sid * rows_per_tile
        pltpu.sync_copy(zrows_hbm.at[pl.ds(r0, rows_per_tile)],
                        agg_sh.at[pl.ds(r0, rows_per_tile)])

        d_src.wait()
        d_dst.wait()
        for b in range(2):
            pltpu.async_copy(x_hbm.at[src_v.at[b]], rows_v.at[b], sem_g[b])
        if flat_counts:
            d_dstf.wait()
        d_zc.wait()
        plsc.subcore_barrier()

        lanes = lax.broadcasted_iota(jnp.int32, (CW,), 0)
        ones16 = jnp.ones((CW,), jnp.float32)

        def count_vec(d16):
            plsc.addupdate_scatter(cnt_v, [d16 * CW + lanes], ones16)

        @pl.loop(0, chunks_per_w, step=2)
        def _chunks(c):
            for b in range(2):
                cc = c + b
                pltpu.make_async_copy(
                    x_hbm.at[src_v.at[cc]], rows_v.at[b], sem_g[b]).wait()
                sdesc = pltpu.async_copy(
                    rows_v.at[b], agg_sh.at[dst_v.at[cc]], sem_s[b],
                    add=True)

                for j in range(cpc):
                    if flat_counts:
                        v = cc * cpc + j

                        @pl.when(v < n_cvecs)
                        def _():
                            count_vec(dstf_v[pl.ds(v * CW, CW)])
                    else:
                        count_vec(dst_v[cc, pl.ds(j * CW, CW)])

                sdesc.wait()

                @pl.when(cc + 2 < chunks_per_w)
                def _():
                    pltpu.async_copy(
                        x_hbm.at[src_v.at[cc + 2]], rows_v.at[b], sem_g[b])

        d_cnt = pltpu.async_copy(cnt_v, cnt_out.at[wid], sem_s[1])

        plsc.subcore_barrier()
        pltpu.sync_copy(agg_sh.at[pl.ds(r0, rows_per_tile)],
                        agg_out.at[cid, pl.ds(r0, rows_per_tile)])
        d_cnt.wait()

    if not flat_counts:
        full = body

        def body_no_flat(src_hbm, dst_hbm, x_hbm, zrows_hbm, zcnt_hbm,
                         agg_out, cnt_out,
                         src_v, dst_v, rows_v, cnt_v, agg_sh, sem_g, sem_s):
            full(src_hbm, dst_hbm, None, x_hbm, zrows_hbm, zcnt_hbm,
                 agg_out, cnt_out,
                 src_v, dst_v, None, rows_v, cnt_v, agg_sh, sem_g, sem_s)

        fn = body_no_flat
    else:
        fn = body

    return pl.kernel(
        fn,
        out_type=(
            jax.ShapeDtypeStruct((NC, n_tgt, D), jnp.float32),
            jax.ShapeDtypeStruct((NW, n_tgt * CW), jnp.float32),
        ),
        mesh=mesh,
        compiler_params=pltpu.CompilerParams(needs_layout_passes=False),
        scratch_types=scratch,
    )


def _tc_layer(aggp, cntp, x_tgt, Wl, Wr, b, a):
    n_tgt = aggp.shape[1]

    def body(agg_ref, cnt_ref, x_ref, wl_ref, wr_ref, b_ref, a_ref, o_ref):
        agg = agg_ref[0] + agg_ref[1]
        cnt = jnp.sum(cnt_ref[...].reshape(NW, n_tgt, CW), axis=(0, 2))[:, None]
        mean = agg / jnp.maximum(cnt, 1.0)
        h = jnp.dot(mean, wl_ref[...], preferred_element_type=jnp.float32)
        h = h + jnp.dot(x_ref[...], wr_ref[...],
                        preferred_element_type=jnp.float32)
        h = h + b_ref[...]
        o_ref[...] = jnp.where(h > 0, h, a_ref[...] * h)

    return pl.pallas_call(
        body,
        out_shape=jax.ShapeDtypeStruct((n_tgt, D), jnp.float32),
        grid=(1,),
        in_specs=[
            pl.BlockSpec((NC, n_tgt, D), lambda i: (0, 0, 0)),
            pl.BlockSpec((NW, n_tgt * CW), lambda i: (0, 0)),
            pl.BlockSpec((n_tgt, D), lambda i: (0, 0)),
            pl.BlockSpec((D, D), lambda i: (0, 0)),
            pl.BlockSpec((D, D), lambda i: (0, 0)),
            pl.BlockSpec((1, D), lambda i: (0, 0)),
            pl.BlockSpec((1, D), lambda i: (0, 0)),
        ],
        out_specs=pl.BlockSpec((n_tgt, D), lambda i: (0, 0)),
    )(aggp, cntp, x_tgt, Wl, Wr, b.reshape(1, D), a.reshape(1, D))


K0 = 125
K1 = 128

_sc_agg0 = _make_sc_agg(N1, E0, K0, flat_counts=True)
_sc_agg1 = _make_sc_agg(N2, E1, K1, flat_counts=False)


def kernel(x, src0, dst0, src1, dst1, Wl0, Wr0, b0, a0, Wl1, Wr1, b1, a1):
    src0 = src0.astype(jnp.int32)
    dst0 = dst0.astype(jnp.int32)
    src1 = src1.astype(jnp.int32)
    dst1 = dst1.astype(jnp.int32)

    z0 = jnp.zeros((N1, D), jnp.float32)
    zc0 = jnp.zeros((N1 * CW,), jnp.float32)
    z1 = jnp.zeros((N2, D), jnp.float32)
    zc1 = jnp.zeros((N2 * CW,), jnp.float32)

    agg0, cnt0 = _sc_agg0(src0.reshape(-1, K0), dst0.reshape(-1, K0),
                          dst0, x, z0, zc0)
    h = _tc_layer(agg0, cnt0, x, Wl0, Wr0, b0, a0)
    agg1, cnt1 = _sc_agg1(src1.reshape(-1, K1), dst1.reshape(-1, K1),
                          h, z1, zc1)
    out = _tc_layer(agg1, cnt1, h, Wl1, Wr1, b1, a1)
    return out
